# baseline (speedup 1.0000x reference)
import jax
import jax.numpy as jnp
from jax.experimental import pallas as pl


def _gat(x, edge_index, W, att_src, att_dst, bias, heads, out_ch, concat):
    n = x.shape[0]
    src = edge_index[0].astype(jnp.int32)
    dst = edge_index[1].astype(jnp.int32)
    loop = jnp.arange(n, dtype=jnp.int32)
    src = jnp.concatenate([src, loop])
    dst = jnp.concatenate([dst, loop])
    h = (x @ W).reshape(n, heads, out_ch)
    alpha_src = (h * att_src[None, :, :]).sum(-1)
    alpha_dst = (h * att_dst[None, :, :]).sum(-1)
    alpha = alpha_src[src] + alpha_dst[dst]
    alpha = jax.nn.leaky_relu(alpha, 0.2)
    amax = jax.ops.segment_max(alpha, dst, num_segments=n)
    amax = jnp.where(jnp.isfinite(amax), amax, 0.0)
    ealpha = jnp.exp(alpha - amax[dst])
    denom = jax.ops.segment_sum(ealpha, dst, num_segments=n)
    alpha = ealpha / (denom[dst] + 1e-16)
    msg = h[src] * alpha[:, :, None]
    out = jax.ops.segment_sum(msg, dst, num_segments=n)
    if concat:
        out = out.reshape(n, heads * out_ch)
    else:
        out = out.mean(axis=1)
    return out + bias


def _bias_add_kernel(x_ref, b_ref, o_ref):
    o_ref[...] = x_ref[...] + b_ref[...]


def kernel(x, edge_attr, edge_index, W1, a_src1, a_dst1, b1, W2, a_src2, a_dst2, b2):
    h = _gat(x, edge_index, W1, a_src1, a_dst1, b1, 8, 128, True)
    h = jax.nn.elu(h)
    out = _gat(h, edge_index, W2, a_src2, a_dst2, jnp.zeros_like(b2), 1, 256, False)
    out = pl.pallas_call(
        _bias_add_kernel,
        out_shape=jax.ShapeDtypeStruct(out.shape, out.dtype),
    )(out, jnp.broadcast_to(b2[None, :], out.shape))
    return out


# speedup vs baseline: 9.9554x; 9.9554x over previous
import functools

import jax
import jax.numpy as jnp
from jax import lax
from jax.experimental import pallas as pl
from jax.experimental.pallas import tpu as pltpu
from jax.experimental.pallas import tpu_sc as plsc

_N = 10000
_E = 160000
_NC = 2
_NS = 16
_NW = _NC * _NS
_BB = 128
_FULL = (_E // (_NW * _BB))
_TAILB = _FULL * _NW * _BB
_STR = _N // _NS
_NEG = -1e30


def _mm_att_body(x_ref, w_ref, asrc_ref, adst_ref,
                 h_ref, asp_ref, adp_ref, gacc_ref, gmax_ref,
                 *, heads, nchunks, nsteps):
    i = pl.program_id(0)
    h = jnp.dot(x_ref[...], w_ref[...], preferred_element_type=jnp.float32)
    r = h.shape[0]
    oc = h.shape[1] // heads
    h3 = h.reshape(r, heads, oc)
    a_s = (h3 * asrc_ref[...][None, :, :]).sum(-1)
    a_d = (h3 * adst_ref[...][None, :, :]).sum(-1)
    pad = jnp.full((r, 16 - heads), _NEG, jnp.float32)
    wide = jnp.zeros((r, 112), jnp.float32)
    asp_ref[...] = jnp.concatenate([a_s, pad, wide], axis=1)
    adp_ref[...] = jnp.concatenate([a_d, pad, wide], axis=1)
    for k in range(nchunks):
        h_ref[k] = h[:, 128 * k:128 * (k + 1)]
    rowpad = jnp.full((1, 16 - heads), _NEG, jnp.float32)
    smax = jnp.concatenate([a_s.max(axis=0, keepdims=True), rowpad], axis=1)
    dmax = jnp.concatenate([a_d.max(axis=0, keepdims=True), rowpad], axis=1)
    cur = jnp.concatenate([smax, dmax], axis=0)

    @pl.when(i == 0)
    def _():
        gacc_ref[...] = cur

    @pl.when(i > 0)
    def _():
        gacc_ref[...] = jnp.maximum(gacc_ref[...], cur)

    @pl.when(i == nsteps - 1)
    def _():
        g = gacc_ref[0:1, :] + gacc_ref[1:2, :]
        g = jnp.where(g > 0, g, 0.2 * g)
        lane = lax.broadcasted_iota(jnp.int32, (1, 16), 1)
        gmax_ref[...] = jnp.where(lane < heads, g, 0.0)


def _mm_att(x, w, asrc, adst, heads, nchunks, blk):
    n, kin = x.shape
    nsteps = n // blk
    body = functools.partial(_mm_att_body, heads=heads, nchunks=nchunks,
                             nsteps=nsteps)
    return pl.pallas_call(
        body,
        grid=(nsteps,),
        in_specs=[
            pl.BlockSpec((blk, kin), lambda i: (i, 0)),
            pl.BlockSpec(w.shape, lambda i: (0, 0)),
            pl.BlockSpec(asrc.shape, lambda i: (0, 0)),
            pl.BlockSpec(adst.shape, lambda i: (0, 0)),
        ],
        out_specs=[
            pl.BlockSpec((nchunks, blk, 128), lambda i: (0, i, 0)),
            pl.BlockSpec((blk, 128), lambda i: (i, 0)),
            pl.BlockSpec((blk, 128), lambda i: (i, 0)),
            pl.BlockSpec((2, 16), lambda i: (0, 0)),
            pl.BlockSpec((1, 16), lambda i: (0, 0)),
        ],
        out_shape=[
            jax.ShapeDtypeStruct((nchunks, n, 128), jnp.float32),
            jax.ShapeDtypeStruct((n, 128), jnp.float32),
            jax.ShapeDtypeStruct((n, 128), jnp.float32),
            jax.ShapeDtypeStruct((2, 16), jnp.float32),
            jax.ShapeDtypeStruct((1, 16), jnp.float32),
        ],
    )(x, w, asrc, adst)


def _sc_edge_body(src_hbm, dst_hbm, asp_hbm, adp_hbm, gmax_hbm, h_hbm,
                  ealpha_hbm, dpart_hbm, upart_hbm,
                  src_v, dst_v, adg, g_v, e_v, rows,
                  u_sh, sem,
                  *, nchunks, lanes):
    c = lax.axis_index("c")
    s = lax.axis_index("s")
    wid = s * _NC + c

    pltpu.sync_copy(gmax_hbm, g_v)
    gvec = g_v[...]

    def edge_rounds(process):
        def body(j, _):
            process((j * _NW + wid) * _BB)
            return 0
        lax.fori_loop(0, _FULL, body, 0)

        @pl.when(wid < 2)
        def _():
            process(jnp.int32(_TAILB) + wid * _BB)

    def zero_rows():
        def zero_r(i, _):
            for jj in range(8):
                rows[i, pl.ds(16 * jj, 16)] = jnp.zeros((16,), jnp.float32)
            return 0
        lax.fori_loop(0, _BB, zero_r, 0)

    def zero_ush():
        for t in range(4):
            pltpu.sync_copy(rows.at[pl.ds(0, _BB)],
                            u_sh.at[pl.ds(s * _STR + t * _BB, _BB)])
        pltpu.sync_copy(rows.at[pl.ds(0, _STR - 4 * _BB)],
                        u_sh.at[pl.ds(s * _STR + 4 * _BB, _STR - 4 * _BB)])

    zero_rows()
    zero_ush()
    plsc.subcore_barrier()

    def pass_a(base):
        pltpu.sync_copy(src_hbm.at[pl.ds(base, _BB)], src_v)
        pltpu.sync_copy(dst_hbm.at[pl.ds(base, _BB)], dst_v)
        pltpu.async_copy(asp_hbm.at[src_v], rows, sem).wait()
        pltpu.async_copy(adp_hbm.at[dst_v], adg, sem).wait()

        def body(i, _):
            a = rows[i, pl.ds(0, 16)] + adg[i, pl.ds(0, 16)]
            a = jnp.where(a > 0, a, 0.2 * a)
            e = jnp.exp(a - gvec)
            e_v[i, :] = e
            rows[i, pl.ds(0, 16)] = e
            for jj in range(1, 8):
                rows[i, pl.ds(16 * jj, 16)] = jnp.zeros((16,), jnp.float32)
            return 0
        lax.fori_loop(0, _BB, body, 0)
        pltpu.sync_copy(e_v, ealpha_hbm.at[pl.ds(base, _BB)])
        pltpu.sync_copy(rows, u_sh.at[dst_v], add=True)

    edge_rounds(pass_a)
    plsc.subcore_barrier()

    @pl.when(s == 0)
    def _():
        pltpu.sync_copy(u_sh, dpart_hbm.at[c])
    plsc.subcore_barrier()

    for k in range(nchunks):
        lane = lanes[k]

        zero_rows()
        zero_ush()
        plsc.subcore_barrier()

        def pass_b(base, _k=k, _lane=lane):
            pltpu.sync_copy(src_hbm.at[pl.ds(base, _BB)], src_v)
            pltpu.sync_copy(dst_hbm.at[pl.ds(base, _BB)], dst_v)
            pltpu.sync_copy(ealpha_hbm.at[pl.ds(base, _BB)], e_v)
            pltpu.async_copy(h_hbm.at[_k].at[src_v], rows, sem).wait()

            def body(i, _):
                ev = e_v[i, :]
                sv = ev.at[jnp.full((16,), _lane, jnp.int32)].get(
                    mode="promise_in_bounds")
                for jj in range(8):
                    sl = pl.ds(16 * jj, 16)
                    rows[i, sl] = rows[i, sl] * sv
                return 0
            lax.fori_loop(0, _BB, body, 0)
            pltpu.sync_copy(rows, u_sh.at[dst_v], add=True)

        edge_rounds(pass_b)
        plsc.subcore_barrier()

        @pl.when(s == 0)
        def _(_k=k):
            pltpu.sync_copy(u_sh, upart_hbm.at[c, _k])
        plsc.subcore_barrier()


def _sc_edge(src, dst, asp, adp, gmaxv, h_heads, nchunks, lanes):
    mesh = plsc.VectorSubcoreMesh(core_axis_name="c", subcore_axis_name="s")
    body = functools.partial(_sc_edge_body, nchunks=nchunks, lanes=lanes)
    fn = functools.partial(
        pl.kernel,
        mesh=mesh,
        out_type=[
            jax.ShapeDtypeStruct((_E, 16), jnp.float32),
            jax.ShapeDtypeStruct((_NC, _N, 128), jnp.float32),
            jax.ShapeDtypeStruct((_NC, nchunks, _N, 128), jnp.float32),
        ],
        scratch_types=[
            pltpu.VMEM((_BB,), jnp.int32),
            pltpu.VMEM((_BB,), jnp.int32),
            pltpu.VMEM((_BB, 128), jnp.float32),
            pltpu.VMEM((16,), jnp.float32),
            pltpu.VMEM((_BB, 16), jnp.float32),
            pltpu.VMEM((_BB, 128), jnp.float32),
            pltpu.VMEM_SHARED((_N, 128), jnp.float32),
            pltpu.SemaphoreType.DMA,
        ],
    )(body)
    return fn(src, dst, asp, adp, gmaxv, h_heads)


def _assemble_body(up_ref, dp_ref, h_ref, asp_ref, adp_ref, gmax_ref, b_ref,
                   o_ref, *, heads, nchunks, lanes, act):
    a = asp_ref[...][:, :heads] + adp_ref[...][:, :heads]
    a = jnp.where(a > 0, a, 0.2 * a)
    eloop = jnp.exp(a - gmax_ref[...][0:1, :heads])
    den = dp_ref[0][:, :heads] + dp_ref[1][:, :heads] + eloop
    cols = []
    for k in range(nchunks):
        ln = lanes[k]
        u = up_ref[0, k] + up_ref[1, k] + eloop[:, ln:ln + 1] * h_ref[k]
        cols.append(u / (den[:, ln:ln + 1] + 1e-16))
    out = jnp.concatenate(cols, axis=1) + b_ref[...]
    if act:
        out = jnp.where(out > 0, out, jnp.exp(out) - 1.0)
    o_ref[...] = out


def _assemble(up, dp, h_heads, asp, adp, gmax, b, heads, nchunks, lanes, act,
              blk):
    nsteps = _N // blk
    body = functools.partial(_assemble_body, heads=heads, nchunks=nchunks,
                             lanes=lanes, act=act)
    return pl.pallas_call(
        body,
        grid=(nsteps,),
        in_specs=[
            pl.BlockSpec((_NC, nchunks, blk, 128), lambda i: (0, 0, i, 0)),
            pl.BlockSpec((_NC, blk, 128), lambda i: (0, i, 0)),
            pl.BlockSpec((nchunks, blk, 128), lambda i: (0, i, 0)),
            pl.BlockSpec((blk, 128), lambda i: (i, 0)),
            pl.BlockSpec((blk, 128), lambda i: (i, 0)),
            pl.BlockSpec((1, 16), lambda i: (0, 0)),
            pl.BlockSpec(b.shape, lambda i: (0, 0)),
        ],
        out_specs=pl.BlockSpec((blk, nchunks * 128), lambda i: (i, 0)),
        out_shape=jax.ShapeDtypeStruct((_N, nchunks * 128), jnp.float32),
    )(up, dp, h_heads, asp, adp, gmax, b)


def kernel(x, edge_attr, edge_index, W1, a_src1, a_dst1, b1,
           W2, a_src2, a_dst2, b2):
    src = edge_index[0].astype(jnp.int32)
    dst = edge_index[1].astype(jnp.int32)

    h1, asp1, adp1, _, gmax1 = _mm_att(x, W1, a_src1, a_dst1,
                                       heads=8, nchunks=8, blk=1000)
    _, d1, u1 = _sc_edge(src, dst, asp1, adp1, gmax1.reshape(16), h1,
                         nchunks=8, lanes=list(range(8)))
    h1o = _assemble(u1, d1, h1, asp1, adp1, gmax1, b1.reshape(1, -1),
                    heads=8, nchunks=8, lanes=list(range(8)), act=True,
                    blk=1000)

    h2, asp2, adp2, _, gmax2 = _mm_att(h1o, W2, a_src2, a_dst2,
                                       heads=1, nchunks=2, blk=1000)
    _, d2, u2 = _sc_edge(src, dst, asp2, adp2, gmax2.reshape(16), h2,
                         nchunks=2, lanes=[0, 0])
    out = _assemble(u2, d2, h2, asp2, adp2, gmax2, b2.reshape(1, -1),
                    heads=1, nchunks=2, lanes=[0, 0], act=False, blk=1000)
    return out
